# baseline (speedup 1.0000x reference)
import dataclasses
import functools

import jax
import jax.numpy as jnp
from jax import lax
from jax.experimental import pallas as pl
from jax.experimental.pallas import tpu as pltpu
from jax.experimental.pallas import tpu_sc as plsc

NSEG = 10000
NROW = 320000
D = 128
NW = 32
RPW = NROW // NW
CH = 400
NCH = NROW // CH
CPW = RPW // CH
STG = 64
LANES = 16
VPR = D // LANES

_MESH = plsc.VectorSubcoreMesh(core_axis_name="c", subcore_axis_name="s")

_COMPILER_PARAMS = pltpu.CompilerParams(use_tc_tiling_on_sc=False)
if "needs_layout_passes" in pltpu.CompilerParams.__dataclass_fields__:
    _COMPILER_PARAMS = dataclasses.replace(_COMPILER_PARAMS,
                                           needs_layout_passes=False)


def _seg_max_body(feats_hbm, batch_hbm, out_hbm, ids_v0, ids_v1, feats_v0,
                  feats_v1, edge_v, stage_v, sems):
    wid = lax.axis_index("c") * 16 + lax.axis_index("s")
    row0 = wid * RPW
    ids_bufs = (ids_v0, ids_v1)
    feats_bufs = (feats_v0, feats_v1)

    lo_off = pl.multiple_of(jnp.maximum(row0 - 8, 0), 8)
    pltpu.sync_copy(batch_hbm.at[pl.ds(lo_off, 8)], edge_v.at[pl.ds(0, 8)])
    hi_off = pl.multiple_of(row0 + RPW - 8, 8)
    pltpu.sync_copy(batch_hbm.at[pl.ds(hi_off, 8)], edge_v.at[pl.ds(8, 8)])
    ev_lo = edge_v[pl.ds(0, 16)]
    own_lo = jnp.where(wid > 0, ev_lo[7] + 1, 0).astype(jnp.int32)
    own_hi = jnp.where(wid < NW - 1, ev_lo[15], NSEG - 1).astype(jnp.int32)

    minf = jnp.full((LANES,), -jnp.inf, jnp.float32)

    def stage_row(vecs, nw, bb):
        slot = nw - bb
        for j in range(VPR):
            stage_v[slot, pl.ds(j * LANES, LANES)] = vecs[j]

        def flush(b):
            pltpu.sync_copy(stage_v, out_hbm.at[pl.ds(b, STG)])
            return b + STG

        bb = lax.cond(slot == STG - 1, flush, lambda b: b, bb)
        return nw + 1, bb

    def fill_to(target, nw, bb):
        def cond(st):
            return st[0] < target

        def body(st):
            return stage_row((minf,) * VPR, st[0], st[1])

        return lax.while_loop(cond, body, (nw, bb))

    iota16 = lax.iota(jnp.int32, LANES)

    def _ffs(mask):
        r = plsc.all_reduce_ffs(mask)
        return r[0] if getattr(r, "ndim", 0) else r

    def _popcnt(mask):
        r = plsc.all_reduce_population_count(mask)
        return r[0] if getattr(r, "ndim", 0) else r

    def make_group_step(ids_v, feats_v):
        def row_vecs(r):
            return [feats_v[r, pl.ds(j * LANES, LANES)] for j in range(VPR)]

        def group_step(g, st):
            cur, nw, bb, acc = st
            base = g * LANES
            idvec = ids_v[pl.ds(base, LANES)]

            def fast(cur, nw, bb, acc):
                new_acc = []
                for j in range(VPR):
                    col = [feats_v[base + k, pl.ds(j * LANES, LANES)]
                           for k in range(LANES)]
                    while len(col) > 1:
                        col = [jnp.maximum(col[i], col[i + 1])
                               for i in range(0, len(col), 2)]
                    new_acc.append(jnp.maximum(acc[j], col[0]))
                return cur, nw, bb, tuple(new_acc)

            def dense(i, a):
                rv = row_vecs(base + i)
                return tuple(jnp.maximum(a[j], rv[j]) for j in range(VPR))

            def sid_at(stop):
                sidv = plsc.load_gather(
                    ids_v, [jnp.full((LANES,), base + stop, jnp.int32)])
                return sidv[0]

            def slow(cur, nw, bb, acc):
                prev_ids = plsc.load_gather(
                    ids_v, [base + jnp.maximum(iota16 - 1, 0)])
                neq = jnp.where(iota16 == 0, idvec != cur,
                                idvec != prev_ids)
                in_own = jnp.logical_and(idvec >= own_lo, idvec <= own_hi)
                newmask = jnp.logical_and(neq, in_own)
                deadmask = idvec > own_hi
                ndead = _popcnt(deadmask)
                n_end = jnp.where(ndead > 0, _ffs(deadmask),
                                  LANES).astype(jnp.int32)
                nb_total = _popcnt(newmask)

                def one_boundary(cur, nw, bb, acc):
                    p = _ffs(newmask)
                    acc = lax.cond(
                        cur >= 0,
                        lambda a: lax.fori_loop(0, p, dense, a),
                        lambda a: a, acc)

                    def close(a, b, acc_=acc):
                        return stage_row(acc_, a, b)

                    nw, bb = lax.cond(cur >= 0, close,
                                      lambda a, b: (a, b), nw, bb)
                    sid = sid_at(p)
                    nw, bb = fill_to(sid, nw, bb)
                    acc = tuple(row_vecs(base + p))
                    acc = lax.fori_loop(p + 1, LANES, dense, acc)
                    return sid, nw, bb, acc

                def general(cur, nw, bb, acc):
                    def rcond(st):
                        return st[0] < n_end

                    def rbody(st):
                        pos, cur, nw, bb, acc = st
                        rest = jnp.logical_and(newmask, iota16 >= pos)
                        nb = jnp.where(_popcnt(rest) > 0, _ffs(rest),
                                       LANES).astype(jnp.int32)
                        stop = jnp.minimum(nb, n_end)

                        acc = lax.cond(
                            cur >= 0,
                            lambda a: lax.fori_loop(pos, stop, dense, a),
                            lambda a: a, acc)

                        def open_new(cur_, nw_, bb_, acc_):
                            def close(a, b):
                                return stage_row(acc_, a, b)

                            nw_, bb_ = lax.cond(cur_ >= 0, close,
                                                lambda a, b: (a, b), nw_,
                                                bb_)
                            sid = sid_at(stop)
                            nw_, bb_ = fill_to(sid, nw_, bb_)
                            return sid, nw_, bb_, tuple(
                                row_vecs(base + stop))

                        cur, nw, bb, acc = lax.cond(
                            stop < n_end, open_new,
                            lambda a, b, c, d: (a, b, c, d), cur, nw, bb,
                            acc)
                        return stop + 1, cur, nw, bb, acc

                    _, cur, nw, bb, acc = lax.while_loop(
                        rcond, rbody, (jnp.int32(0), cur, nw, bb, acc))
                    return cur, nw, bb, acc

                streamlined = jnp.logical_and(nb_total == 1, ndead == 0)
                return lax.cond(streamlined, one_boundary, general, cur,
                                nw, bb, acc)

            all_cont = _popcnt(idvec == cur) == LANES
            return lax.cond(all_cont, fast, slow, cur, nw, bb, acc)

        def pair_step(p, st):
            cur, nw, bb, acc = st
            base = p * 2 * LANES
            id0 = ids_v[pl.ds(base, LANES)]
            id1 = ids_v[pl.ds(base + LANES, LANES)]
            n_same = _popcnt(id0 == cur) + _popcnt(id1 == cur)

            def fast32(cur, nw, bb, acc):
                new_acc = []
                for j in range(VPR):
                    col = [feats_v[base + k, pl.ds(j * LANES, LANES)]
                           for k in range(2 * LANES)]
                    while len(col) > 1:
                        col = [jnp.maximum(col[i], col[i + 1])
                               for i in range(0, len(col), 2)]
                    new_acc.append(jnp.maximum(acc[j], col[0]))
                return cur, nw, bb, tuple(new_acc)

            def two_groups(cur, nw, bb, acc):
                st = group_step(2 * p, (cur, nw, bb, acc))
                return group_step(2 * p + 1, st)

            return lax.cond(n_same == 2 * LANES, fast32, two_groups, cur,
                            nw, bb, acc)

        return group_step, pair_step

    def issue(c, slot):
        base = pl.multiple_of(c * CH, 8)
        pltpu.async_copy(batch_hbm.at[pl.ds(base, CH)], ids_bufs[slot],
                         sems.at[slot])
        pltpu.async_copy(feats_hbm.at[pl.ds(base, CH)], feats_bufs[slot],
                         sems.at[2 + slot])

    def wait_slot(slot):
        pltpu.make_async_copy(batch_hbm.at[pl.ds(0, CH)], ids_bufs[slot],
                              sems.at[slot]).wait()
        pltpu.make_async_copy(feats_hbm.at[pl.ds(0, CH)], feats_bufs[slot],
                              sems.at[2 + slot]).wait()

    c0 = jnp.int32(wid * CPW)

    def make_chunk_body(slot):
        def f(c, cur, nw, bb, acc):
            issue(jnp.minimum(c + 1, NCH - 1), 1 - slot)
            wait_slot(slot)
            group_step, pair_step = make_group_step(ids_bufs[slot],
                                                    feats_bufs[slot])
            st = lax.fori_loop(0, (CH // LANES) // 2, pair_step,
                               (cur, nw, bb, acc))
            cur, nw, bb, acc = group_step((CH // LANES) - 1, st)
            idtail = ids_bufs[slot][pl.ds(CH - LANES, LANES)]
            cont = jnp.logical_and(idtail[LANES - 1] <= own_hi,
                                   c + 1 < NCH)
            return cont, cur, nw, bb, acc

        return f

    def outer_cond(st):
        return st[1]

    def outer_body(st):
        k, _, cur, nw, bb, acc = st
        c = c0 + k
        cont, cur, nw, bb, acc = lax.cond((k & 1) == 0, make_chunk_body(0),
                                          make_chunk_body(1), c, cur, nw,
                                          bb, acc)
        return k + 1, cont, cur, nw, bb, acc

    acc0 = (minf,) * VPR
    issue(c0, 0)
    init = (jnp.int32(0), jnp.bool_(True), jnp.int32(-1), own_lo, own_lo,
            acc0)
    k_fin, _, cur, nw, bb, acc = lax.while_loop(outer_cond, outer_body,
                                                init)
    lax.cond((k_fin & 1) == 0, lambda: wait_slot(0), lambda: wait_slot(1))

    def close_fin(a, b):
        return stage_row(acc, a, b)

    nw, bb = lax.cond(cur >= 0, close_fin, lambda a, b: (a, b), nw, bb)
    nw, bb = fill_to(own_hi + 1, nw, bb)

    rem = nw - bb
    off = jnp.int32(0)
    for sz in (32, 16, 8, 4, 2, 1):
        def do(o, _sz=sz):
            pltpu.sync_copy(stage_v.at[pl.ds(o, _sz)],
                            out_hbm.at[pl.ds(bb + o, _sz)])
            return o + _sz

        off = lax.cond((rem & sz) != 0, do, lambda o: o, off)


@functools.partial(pl.kernel,
                   out_type=jax.ShapeDtypeStruct((NSEG, D), jnp.float32),
                   mesh=_MESH,
                   compiler_params=_COMPILER_PARAMS,
                   scratch_types=[
                       pltpu.VMEM((CH,), jnp.int32),
                       pltpu.VMEM((CH,), jnp.int32),
                       pltpu.VMEM((CH, D), jnp.float32),
                       pltpu.VMEM((CH, D), jnp.float32),
                       pltpu.VMEM((16,), jnp.int32),
                       pltpu.VMEM((STG, D), jnp.float32),
                       pltpu.SemaphoreType.DMA((4,)),
                   ])
def _seg_max_kernel(feats_hbm, batch_hbm, out_hbm, ids_v0, ids_v1,
                    feats_v0, feats_v1, edge_v, stage_v, sems):
    _seg_max_body(feats_hbm, batch_hbm, out_hbm, ids_v0, ids_v1, feats_v0,
                  feats_v1, edge_v, stage_v, sems)


def kernel(feats, batch):
    return _seg_max_kernel(feats, batch.astype(jnp.int32))


# speedup vs baseline: 1.4915x; 1.4915x over previous
import dataclasses
import functools

import jax
import jax.numpy as jnp
from jax import lax
from jax.experimental import pallas as pl
from jax.experimental.pallas import tpu as pltpu
from jax.experimental.pallas import tpu_sc as plsc

NSEG = 10000
NROW = 320000
D = 128
NW = 32
RPW = NROW // NW
CH = 400
NCH = NROW // CH
CPW = RPW // CH
STG = 64
LANES = 16
VPR = D // LANES

_MESH = plsc.VectorSubcoreMesh(core_axis_name="c", subcore_axis_name="s")

_COMPILER_PARAMS = pltpu.CompilerParams(use_tc_tiling_on_sc=False)
if "needs_layout_passes" in pltpu.CompilerParams.__dataclass_fields__:
    _COMPILER_PARAMS = dataclasses.replace(_COMPILER_PARAMS,
                                           needs_layout_passes=False)


def _seg_max_body(feats_hbm, batch_hbm, out_hbm, ids_v0, ids_v1, feats_v0,
                  feats_v1, edge_v, stage_v, sems):
    wid = lax.axis_index("c") * 16 + lax.axis_index("s")
    row0 = wid * RPW
    ids_bufs = (ids_v0, ids_v1)
    feats_bufs = (feats_v0, feats_v1)

    lo_off = pl.multiple_of(jnp.maximum(row0 - 8, 0), 8)
    pltpu.sync_copy(batch_hbm.at[pl.ds(lo_off, 8)], edge_v.at[pl.ds(0, 8)])
    hi_off = pl.multiple_of(row0 + RPW - 8, 8)
    pltpu.sync_copy(batch_hbm.at[pl.ds(hi_off, 8)], edge_v.at[pl.ds(8, 8)])
    ev_lo = edge_v[pl.ds(0, 16)]
    own_lo = jnp.where(wid > 0, ev_lo[7] + 1, 0).astype(jnp.int32)
    own_hi = jnp.where(wid < NW - 1, ev_lo[15], NSEG - 1).astype(jnp.int32)

    minf = jnp.full((LANES,), -jnp.inf, jnp.float32)

    def stage_row(vecs, nw, bb):
        slot = nw - bb
        for j in range(VPR):
            stage_v[slot, pl.ds(j * LANES, LANES)] = vecs[j]

        def flush(b):
            pltpu.sync_copy(stage_v, out_hbm.at[pl.ds(b, STG)])
            return b + STG

        bb = lax.cond(slot == STG - 1, flush, lambda b: b, bb)
        return nw + 1, bb

    def fill_to(target, nw, bb):
        def cond(st):
            return st[0] < target

        def body(st):
            return stage_row((minf,) * VPR, st[0], st[1])

        return lax.while_loop(cond, body, (nw, bb))

    iota16 = lax.iota(jnp.int32, LANES)

    def _ffs(mask):
        r = plsc.all_reduce_ffs(mask)
        return r[0] if getattr(r, "ndim", 0) else r

    def _popcnt(mask):
        r = plsc.all_reduce_population_count(mask)
        return r[0] if getattr(r, "ndim", 0) else r

    def make_group_step(ids_v, feats_v):
        def row_vecs(r):
            return [feats_v[r, pl.ds(j * LANES, LANES)] for j in range(VPR)]

        def group_step(g, st):
            cur, nw, bb, acc = st
            base = g * LANES
            idvec = ids_v[pl.ds(base, LANES)]

            def fast(cur, nw, bb, acc):
                new_acc = []
                for j in range(VPR):
                    col = [feats_v[base + k, pl.ds(j * LANES, LANES)]
                           for k in range(LANES)]
                    while len(col) > 1:
                        col = [jnp.maximum(col[i], col[i + 1])
                               for i in range(0, len(col), 2)]
                    new_acc.append(jnp.maximum(acc[j], col[0]))
                return cur, nw, bb, tuple(new_acc)

            def dense(i, a):
                rv = row_vecs(base + i)
                return tuple(jnp.maximum(a[j], rv[j]) for j in range(VPR))

            def sid_at(stop):
                sidv = plsc.load_gather(
                    ids_v, [jnp.full((LANES,), base + stop, jnp.int32)])
                return sidv[0]

            def slow(cur, nw, bb, acc):
                prev_ids = plsc.load_gather(
                    ids_v, [base + jnp.maximum(iota16 - 1, 0)])
                neq = jnp.where(iota16 == 0, idvec != cur,
                                idvec != prev_ids)
                in_own = jnp.logical_and(idvec >= own_lo, idvec <= own_hi)
                newmask = jnp.logical_and(neq, in_own)
                deadmask = idvec > own_hi
                ndead = _popcnt(deadmask)
                n_end = jnp.where(ndead > 0, _ffs(deadmask),
                                  LANES).astype(jnp.int32)
                nb_total = _popcnt(newmask)

                def one_boundary(cur, nw, bb, acc):
                    p = _ffs(newmask)
                    acc = lax.cond(
                        cur >= 0,
                        lambda a: lax.fori_loop(0, p, dense, a),
                        lambda a: a, acc)

                    def close(a, b, acc_=acc):
                        return stage_row(acc_, a, b)

                    nw, bb = lax.cond(cur >= 0, close,
                                      lambda a, b: (a, b), nw, bb)
                    sid = sid_at(p)
                    nw, bb = fill_to(sid, nw, bb)
                    acc = tuple(row_vecs(base + p))
                    acc = lax.fori_loop(p + 1, LANES, dense, acc)
                    return sid, nw, bb, acc

                def general(cur, nw, bb, acc):
                    def rcond(st):
                        return st[0] < n_end

                    def rbody(st):
                        pos, cur, nw, bb, acc = st
                        rest = jnp.logical_and(newmask, iota16 >= pos)
                        nb = jnp.where(_popcnt(rest) > 0, _ffs(rest),
                                       LANES).astype(jnp.int32)
                        stop = jnp.minimum(nb, n_end)

                        acc = lax.cond(
                            cur >= 0,
                            lambda a: lax.fori_loop(pos, stop, dense, a),
                            lambda a: a, acc)

                        def open_new(cur_, nw_, bb_, acc_):
                            def close(a, b):
                                return stage_row(acc_, a, b)

                            nw_, bb_ = lax.cond(cur_ >= 0, close,
                                                lambda a, b: (a, b), nw_,
                                                bb_)
                            sid = sid_at(stop)
                            nw_, bb_ = fill_to(sid, nw_, bb_)
                            return sid, nw_, bb_, tuple(
                                row_vecs(base + stop))

                        cur, nw, bb, acc = lax.cond(
                            stop < n_end, open_new,
                            lambda a, b, c, d: (a, b, c, d), cur, nw, bb,
                            acc)
                        return stop + 1, cur, nw, bb, acc

                    _, cur, nw, bb, acc = lax.while_loop(
                        rcond, rbody, (jnp.int32(0), cur, nw, bb, acc))
                    return cur, nw, bb, acc

                streamlined = jnp.logical_and(nb_total == 1, ndead == 0)
                return lax.cond(streamlined, one_boundary, general, cur,
                                nw, bb, acc)

            all_cont = idvec[LANES - 1] == cur
            return lax.cond(all_cont, fast, slow, cur, nw, bb, acc)

        return group_step

    def issue(c, slot):
        base = pl.multiple_of(c * CH, 8)
        pltpu.async_copy(batch_hbm.at[pl.ds(base, CH)], ids_bufs[slot],
                         sems.at[slot])
        pltpu.async_copy(feats_hbm.at[pl.ds(base, CH)], feats_bufs[slot],
                         sems.at[2 + slot])

    def wait_slot(slot):
        pltpu.make_async_copy(batch_hbm.at[pl.ds(0, CH)], ids_bufs[slot],
                              sems.at[slot]).wait()
        pltpu.make_async_copy(feats_hbm.at[pl.ds(0, CH)], feats_bufs[slot],
                              sems.at[2 + slot]).wait()

    c0 = jnp.int32(wid * CPW)

    def make_chunk_body(slot):
        def f(c, cur, nw, bb, acc):
            issue(jnp.minimum(c + 1, NCH - 1), 1 - slot)
            wait_slot(slot)
            cur, nw, bb, acc = lax.fori_loop(
                0, CH // LANES, make_group_step(ids_bufs[slot],
                                                feats_bufs[slot]),
                (cur, nw, bb, acc))
            idtail = ids_bufs[slot][pl.ds(CH - LANES, LANES)]
            cont = jnp.logical_and(idtail[LANES - 1] <= own_hi,
                                   c + 1 < NCH)
            return cont, cur, nw, bb, acc

        return f

    def outer_cond(st):
        return st[1]

    def outer_body(st):
        k, _, cur, nw, bb, acc = st
        c = c0 + k
        cont, cur, nw, bb, acc = lax.cond((k & 1) == 0, make_chunk_body(0),
                                          make_chunk_body(1), c, cur, nw,
                                          bb, acc)
        return k + 1, cont, cur, nw, bb, acc

    acc0 = (minf,) * VPR
    issue(c0, 0)
    init = (jnp.int32(0), jnp.bool_(True), jnp.int32(-1), own_lo, own_lo,
            acc0)
    k_fin, _, cur, nw, bb, acc = lax.while_loop(outer_cond, outer_body,
                                                init)
    lax.cond((k_fin & 1) == 0, lambda: wait_slot(0), lambda: wait_slot(1))

    def close_fin(a, b):
        return stage_row(acc, a, b)

    nw, bb = lax.cond(cur >= 0, close_fin, lambda a, b: (a, b), nw, bb)
    nw, bb = fill_to(own_hi + 1, nw, bb)

    rem = nw - bb
    off = jnp.int32(0)
    for sz in (32, 16, 8, 4, 2, 1):
        def do(o, _sz=sz):
            pltpu.sync_copy(stage_v.at[pl.ds(o, _sz)],
                            out_hbm.at[pl.ds(bb + o, _sz)])
            return o + _sz

        off = lax.cond((rem & sz) != 0, do, lambda o: o, off)


@functools.partial(pl.kernel,
                   out_type=jax.ShapeDtypeStruct((NSEG, D), jnp.float32),
                   mesh=_MESH,
                   compiler_params=_COMPILER_PARAMS,
                   scratch_types=[
                       pltpu.VMEM((CH,), jnp.int32),
                       pltpu.VMEM((CH,), jnp.int32),
                       pltpu.VMEM((CH, D), jnp.float32),
                       pltpu.VMEM((CH, D), jnp.float32),
                       pltpu.VMEM((16,), jnp.int32),
                       pltpu.VMEM((STG, D), jnp.float32),
                       pltpu.SemaphoreType.DMA((4,)),
                   ])
def _seg_max_kernel(feats_hbm, batch_hbm, out_hbm, ids_v0, ids_v1,
                    feats_v0, feats_v1, edge_v, stage_v, sems):
    _seg_max_body(feats_hbm, batch_hbm, out_hbm, ids_v0, ids_v1, feats_v0,
                  feats_v1, edge_v, stage_v, sems)


def kernel(feats, batch):
    return _seg_max_kernel(feats, batch.astype(jnp.int32))


# speedup vs baseline: 1.5040x; 1.0084x over previous
import dataclasses
import functools

import jax
import jax.numpy as jnp
from jax import lax
from jax.experimental import pallas as pl
from jax.experimental.pallas import tpu as pltpu
from jax.experimental.pallas import tpu_sc as plsc

NSEG = 10000
NROW = 320000
D = 128
NW = 32
RPW = NROW // NW
CH = 400
NCH = NROW // CH
CPW = RPW // CH
STG = 64
LANES = 16
VPR = D // LANES

_MESH = plsc.VectorSubcoreMesh(core_axis_name="c", subcore_axis_name="s")

_COMPILER_PARAMS = pltpu.CompilerParams(use_tc_tiling_on_sc=False)
if "needs_layout_passes" in pltpu.CompilerParams.__dataclass_fields__:
    _COMPILER_PARAMS = dataclasses.replace(_COMPILER_PARAMS,
                                           needs_layout_passes=False)


def _seg_max_body(feats_hbm, batch_hbm, out_hbm, ids_v0, ids_v1, feats_v0,
                  feats_v1, edge_v, stage_v, sems):
    wid = lax.axis_index("c") * 16 + lax.axis_index("s")
    row0 = wid * RPW
    ids_bufs = (ids_v0, ids_v1)
    feats_bufs = (feats_v0, feats_v1)

    lo_off = pl.multiple_of(jnp.maximum(row0 - 8, 0), 8)
    pltpu.sync_copy(batch_hbm.at[pl.ds(lo_off, 8)], edge_v.at[pl.ds(0, 8)])
    hi_off = pl.multiple_of(row0 + RPW - 8, 8)
    pltpu.sync_copy(batch_hbm.at[pl.ds(hi_off, 8)], edge_v.at[pl.ds(8, 8)])
    ev_lo = edge_v[pl.ds(0, 16)]
    own_lo = jnp.where(wid > 0, ev_lo[7] + 1, 0).astype(jnp.int32)
    own_hi = jnp.where(wid < NW - 1, ev_lo[15], NSEG - 1).astype(jnp.int32)

    minf = jnp.full((LANES,), -jnp.inf, jnp.float32)

    def _flush_issue(half, b):
        pltpu.async_copy(stage_v.at[pl.ds(half * STG, STG)],
                         out_hbm.at[pl.ds(b, STG)], sems.at[4 + half])

    def _flush_wait(half):
        pltpu.make_async_copy(stage_v.at[pl.ds(half * STG, STG)],
                              out_hbm.at[pl.ds(0, STG)],
                              sems.at[4 + half]).wait()

    def stage_row(vecs, nw, bb):
        slot = nw - bb
        bufp = lax.shift_right_logical(bb - own_lo, 6) & 1
        for j in range(VPR):
            stage_v[slot + bufp * STG, pl.ds(j * LANES, LANES)] = vecs[j]

        def flush(b):
            lax.cond(bufp == 0, lambda: _flush_issue(0, b),
                     lambda: _flush_issue(1, b))
            lax.cond(b - own_lo >= STG,
                     lambda: lax.cond(bufp == 0, lambda: _flush_wait(1),
                                      lambda: _flush_wait(0)),
                     lambda: None)
            return b + STG

        bb = lax.cond(slot == STG - 1, flush, lambda b: b, bb)
        return nw + 1, bb

    def fill_to(target, nw, bb):
        def cond(st):
            return st[0] < target

        def body(st):
            return stage_row((minf,) * VPR, st[0], st[1])

        return lax.while_loop(cond, body, (nw, bb))

    iota16 = lax.iota(jnp.int32, LANES)

    def _ffs(mask):
        r = plsc.all_reduce_ffs(mask)
        return r[0] if getattr(r, "ndim", 0) else r

    def _popcnt(mask):
        r = plsc.all_reduce_population_count(mask)
        return r[0] if getattr(r, "ndim", 0) else r

    def make_group_step(ids_v, feats_v):
        def row_vecs(r):
            return [feats_v[r, pl.ds(j * LANES, LANES)] for j in range(VPR)]

        def group_step(g, st):
            cur, nw, bb, acc = st
            base = g * LANES
            idvec = ids_v[pl.ds(base, LANES)]

            def fast(cur, nw, bb, acc):
                new_acc = []
                for j in range(VPR):
                    col = [feats_v[base + k, pl.ds(j * LANES, LANES)]
                           for k in range(LANES)]
                    while len(col) > 1:
                        col = [jnp.maximum(col[i], col[i + 1])
                               for i in range(0, len(col), 2)]
                    new_acc.append(jnp.maximum(acc[j], col[0]))
                return cur, nw, bb, tuple(new_acc)

            def dense(i, a):
                rv = row_vecs(base + i)
                return tuple(jnp.maximum(a[j], rv[j]) for j in range(VPR))

            def sid_at(stop):
                sidv = plsc.load_gather(
                    ids_v, [jnp.full((LANES,), base + stop, jnp.int32)])
                return sidv[0]

            def slow(cur, nw, bb, acc):
                prev_ids = plsc.load_gather(
                    ids_v, [base + jnp.maximum(iota16 - 1, 0)])
                neq = jnp.where(iota16 == 0, idvec != cur,
                                idvec != prev_ids)
                in_own = jnp.logical_and(idvec >= own_lo, idvec <= own_hi)
                newmask = jnp.logical_and(neq, in_own)
                deadmask = idvec > own_hi
                ndead = _popcnt(deadmask)
                n_end = jnp.where(ndead > 0, _ffs(deadmask),
                                  LANES).astype(jnp.int32)
                nb_total = _popcnt(newmask)

                def one_boundary(cur, nw, bb, acc):
                    p = _ffs(newmask)
                    acc = lax.cond(
                        cur >= 0,
                        lambda a: lax.fori_loop(0, p, dense, a),
                        lambda a: a, acc)

                    def close(a, b, acc_=acc):
                        return stage_row(acc_, a, b)

                    nw, bb = lax.cond(cur >= 0, close,
                                      lambda a, b: (a, b), nw, bb)
                    sid = sid_at(p)
                    nw, bb = fill_to(sid, nw, bb)
                    acc = tuple(row_vecs(base + p))
                    acc = lax.fori_loop(p + 1, LANES, dense, acc)
                    return sid, nw, bb, acc

                def general(cur, nw, bb, acc):
                    def rcond(st):
                        return st[0] < n_end

                    def rbody(st):
                        pos, cur, nw, bb, acc = st
                        rest = jnp.logical_and(newmask, iota16 >= pos)
                        nb = jnp.where(_popcnt(rest) > 0, _ffs(rest),
                                       LANES).astype(jnp.int32)
                        stop = jnp.minimum(nb, n_end)

                        acc = lax.cond(
                            cur >= 0,
                            lambda a: lax.fori_loop(pos, stop, dense, a),
                            lambda a: a, acc)

                        def open_new(cur_, nw_, bb_, acc_):
                            def close(a, b):
                                return stage_row(acc_, a, b)

                            nw_, bb_ = lax.cond(cur_ >= 0, close,
                                                lambda a, b: (a, b), nw_,
                                                bb_)
                            sid = sid_at(stop)
                            nw_, bb_ = fill_to(sid, nw_, bb_)
                            return sid, nw_, bb_, tuple(
                                row_vecs(base + stop))

                        cur, nw, bb, acc = lax.cond(
                            stop < n_end, open_new,
                            lambda a, b, c, d: (a, b, c, d), cur, nw, bb,
                            acc)
                        return stop + 1, cur, nw, bb, acc

                    _, cur, nw, bb, acc = lax.while_loop(
                        rcond, rbody, (jnp.int32(0), cur, nw, bb, acc))
                    return cur, nw, bb, acc

                streamlined = jnp.logical_and(nb_total == 1, ndead == 0)
                return lax.cond(streamlined, one_boundary, general, cur,
                                nw, bb, acc)

            all_cont = idvec[LANES - 1] == cur
            return lax.cond(all_cont, fast, slow, cur, nw, bb, acc)

        return group_step

    def issue(c, slot):
        base = pl.multiple_of(c * CH, 8)
        pltpu.async_copy(batch_hbm.at[pl.ds(base, CH)], ids_bufs[slot],
                         sems.at[slot])
        pltpu.async_copy(feats_hbm.at[pl.ds(base, CH)], feats_bufs[slot],
                         sems.at[2 + slot])

    def wait_slot(slot):
        pltpu.make_async_copy(batch_hbm.at[pl.ds(0, CH)], ids_bufs[slot],
                              sems.at[slot]).wait()
        pltpu.make_async_copy(feats_hbm.at[pl.ds(0, CH)], feats_bufs[slot],
                              sems.at[2 + slot]).wait()

    c0 = jnp.int32(wid * CPW)

    def make_chunk_body(slot):
        def f(c, cur, nw, bb, acc):
            issue(jnp.minimum(c + 1, NCH - 1), 1 - slot)
            wait_slot(slot)
            cur, nw, bb, acc = lax.fori_loop(
                0, CH // LANES, make_group_step(ids_bufs[slot],
                                                feats_bufs[slot]),
                (cur, nw, bb, acc))
            idtail = ids_bufs[slot][pl.ds(CH - LANES, LANES)]
            cont = jnp.logical_and(idtail[LANES - 1] <= own_hi,
                                   c + 1 < NCH)
            return cont, cur, nw, bb, acc

        return f

    def outer_cond(st):
        return st[1]

    def outer_body(st):
        k, _, cur, nw, bb, acc = st
        c = c0 + k
        cont, cur, nw, bb, acc = lax.cond((k & 1) == 0, make_chunk_body(0),
                                          make_chunk_body(1), c, cur, nw,
                                          bb, acc)
        return k + 1, cont, cur, nw, bb, acc

    acc0 = (minf,) * VPR
    issue(c0, 0)
    init = (jnp.int32(0), jnp.bool_(True), jnp.int32(-1), own_lo, own_lo,
            acc0)
    k_fin, _, cur, nw, bb, acc = lax.while_loop(outer_cond, outer_body,
                                                init)
    lax.cond((k_fin & 1) == 0, lambda: wait_slot(0), lambda: wait_slot(1))

    def close_fin(a, b):
        return stage_row(acc, a, b)

    nw, bb = lax.cond(cur >= 0, close_fin, lambda a, b: (a, b), nw, bb)
    nw, bb = fill_to(own_hi + 1, nw, bb)

    nflushed = lax.shift_right_logical(bb - own_lo, 6)
    lax.cond(nflushed >= 1,
             lambda: lax.cond((nflushed - 1) & 1 == 0,
                              lambda: _flush_wait(0),
                              lambda: _flush_wait(1)),
             lambda: None)
    srcoff = (nflushed & 1) * STG
    rem = nw - bb
    off = jnp.int32(0)
    for sz in (32, 16, 8, 4, 2, 1):
        def do(o, _sz=sz):
            pltpu.sync_copy(stage_v.at[pl.ds(srcoff + o, _sz)],
                            out_hbm.at[pl.ds(bb + o, _sz)])
            return o + _sz

        off = lax.cond((rem & sz) != 0, do, lambda o: o, off)


@functools.partial(pl.kernel,
                   out_type=jax.ShapeDtypeStruct((NSEG, D), jnp.float32),
                   mesh=_MESH,
                   compiler_params=_COMPILER_PARAMS,
                   scratch_types=[
                       pltpu.VMEM((CH,), jnp.int32),
                       pltpu.VMEM((CH,), jnp.int32),
                       pltpu.VMEM((CH, D), jnp.float32),
                       pltpu.VMEM((CH, D), jnp.float32),
                       pltpu.VMEM((16,), jnp.int32),
                       pltpu.VMEM((2 * STG, D), jnp.float32),
                       pltpu.SemaphoreType.DMA((6,)),
                   ])
def _seg_max_kernel(feats_hbm, batch_hbm, out_hbm, ids_v0, ids_v1,
                    feats_v0, feats_v1, edge_v, stage_v, sems):
    _seg_max_body(feats_hbm, batch_hbm, out_hbm, ids_v0, ids_v1, feats_v0,
                  feats_v1, edge_v, stage_v, sems)


def kernel(feats, batch):
    return _seg_max_kernel(feats, batch.astype(jnp.int32))
